# baseline (speedup 1.0000x reference)
import numpy as np

import jax
import jax.numpy as jnp
from jax.experimental import pallas as pl
from jax.sharding import Mesh, PartitionSpec as P

try:
    from jax import shard_map as _shard_map

    def _smap(f, mesh, in_specs, out_specs):
        return _shard_map(
            f, mesh=mesh, in_specs=in_specs, out_specs=out_specs, check_vma=False
        )
except ImportError:
    from jax.experimental.shard_map import shard_map as _shard_map_old

    def _smap(f, mesh, in_specs, out_specs):
        return _shard_map_old(
            f, mesh=mesh, in_specs=in_specs, out_specs=out_specs, check_rep=False
        )

_S_BLK = 2048


def _body(x_ref, pe_ref, o_ref):
    o_ref[...] = x_ref[...] + pe_ref[...]


def _add(x, pe):
    B, S, E = x.shape
    s_blk = min(_S_BLK, S)
    grid = (S // s_blk, B)
    return pl.pallas_call(
        _body,
        grid=grid,
        in_specs=[
            pl.BlockSpec((1, s_blk, E), lambda i, b: (b, i, 0)),
            pl.BlockSpec((s_blk, E), lambda i, b: (i, 0)),
        ],
        out_specs=pl.BlockSpec((1, s_blk, E), lambda i, b: (b, i, 0)),
        out_shape=jax.ShapeDtypeStruct(x.shape, x.dtype),
    )(x, pe)


def kernel(x, pe):
    B, S, E = x.shape
    devs = jax.devices()
    if len(devs) >= 2 and S % (2 * _S_BLK) == 0:
        mesh = Mesh(np.asarray(devs[:2]), ("d",))
        out = _smap(
            _add,
            mesh,
            (P(None, "d", None), P("d", None)),
            P(None, "d", None),
        )(x, pe[:S])
        return jax.device_put(out, devs[0])
    return _add(x, pe)


# speedup vs baseline: 10.2784x; 10.2784x over previous
import jax
import jax.numpy as jnp
from jax.experimental import pallas as pl
from jax.experimental.pallas import tpu as pltpu

_R = 1024
_K = 4
_PK = 2


def _body(x_hbm, pe_hbm, o_hbm, xbuf, pebuf, obuf, xsem, pesem, osem):
    B, S, E = x_hbm.shape
    ns = S // _R
    T = ns * B

    def start_x(t):
        i, b = divmod(t, B)
        pltpu.make_async_copy(
            x_hbm.at[b, pl.ds(i * _R, _R), :], xbuf.at[t % _K], xsem.at[t % _K]
        ).start()

    def wait_x(t):
        i, b = divmod(t, B)
        pltpu.make_async_copy(
            x_hbm.at[b, pl.ds(i * _R, _R), :], xbuf.at[t % _K], xsem.at[t % _K]
        ).wait()

    def start_pe(i):
        pltpu.make_async_copy(
            pe_hbm.at[pl.ds(i * _R, _R), :], pebuf.at[i % _PK], pesem.at[i % _PK]
        ).start()

    def wait_pe(i):
        pltpu.make_async_copy(
            pe_hbm.at[pl.ds(i * _R, _R), :], pebuf.at[i % _PK], pesem.at[i % _PK]
        ).wait()

    def start_out(t):
        i, b = divmod(t, B)
        pltpu.make_async_copy(
            obuf.at[t % _K], o_hbm.at[b, pl.ds(i * _R, _R), :], osem.at[t % _K]
        ).start()

    def wait_out(t):
        i, b = divmod(t, B)
        pltpu.make_async_copy(
            obuf.at[t % _K], o_hbm.at[b, pl.ds(i * _R, _R), :], osem.at[t % _K]
        ).wait()

    for t in range(min(_K - 1, T)):
        start_x(t)
    for i in range(min(_PK, ns)):
        start_pe(i)

    for t in range(T):
        i, b = divmod(t, B)
        wait_x(t)
        if b == 0:
            wait_pe(i)
        if t >= _K:
            wait_out(t - _K)
        obuf[t % _K] = xbuf[t % _K][...] + pebuf[i % _PK][...]
        start_out(t)
        if t + _K - 1 < T:
            start_x(t + _K - 1)
        if b == B - 1 and i + _PK < ns:
            start_pe(i + _PK)

    for t in range(max(T - _K, 0), T):
        wait_out(t)


def kernel(x, pe):
    B, S, E = x.shape
    return pl.pallas_call(
        _body,
        in_specs=[
            pl.BlockSpec(memory_space=pltpu.HBM),
            pl.BlockSpec(memory_space=pltpu.HBM),
        ],
        out_specs=pl.BlockSpec(memory_space=pltpu.HBM),
        out_shape=jax.ShapeDtypeStruct(x.shape, x.dtype),
        scratch_shapes=[
            pltpu.VMEM((_K, _R, E), x.dtype),
            pltpu.VMEM((_PK, _R, E), pe.dtype),
            pltpu.VMEM((_K, _R, E), x.dtype),
            pltpu.SemaphoreType.DMA((_K,)),
            pltpu.SemaphoreType.DMA((_PK,)),
            pltpu.SemaphoreType.DMA((_K,)),
        ],
    )(x, pe)
